# Initial kernel scaffold; baseline (speedup 1.0000x reference)
import jax
import jax.numpy as jnp
from jax.experimental import pallas as pl


def kernel(rel_probs, edge_index, num_nodes):
    raise NotImplementedError("write your pallas kernel here")


# speedup vs baseline: 3.2375x; 3.2375x over previous
import functools

import jax
import jax.numpy as jnp
from jax import lax
from jax.experimental import pallas as pl
from jax.experimental.pallas import tpu as pltpu
from jax.experimental.pallas import tpu_sc as plsc

N_NODES = 4096
E = 262144
NREL = 8
PARENT = 4
SEQUENCE = 6

NC = 2
NS = 16
NW = NC * NS
CH = E // NW
C128 = CH // 128
KG = 16
TBL = N_NODES * N_NODES


def _mesh():
    return plsc.VectorSubcoreMesh(core_axis_name="c", subcore_axis_name="s")


_PREP_COLS = (0, 1, 2, 3, SEQUENCE, PARENT)


def _prep_body(rel_ref, *out_refs):
    x = rel_ref[...]
    for c, o in zip(_PREP_COLS, out_refs):
        o[...] = x[:, :, c]


_prep = pl.pallas_call(
    _prep_body,
    grid=(E // (8 * 128),),
    in_specs=[pl.BlockSpec((8, 128, NREL), lambda i: (i, 0, 0))],
    out_specs=[pl.BlockSpec((8, 128), lambda i: (i, 0))] * len(_PREP_COLS),
    out_shape=[jax.ShapeDtypeStruct((E // 128, 128), jnp.float32)]
    * len(_PREP_COLS),
)


@functools.partial(
    pl.kernel,
    mesh=_mesh(),
    out_type=[
        jax.ShapeDtypeStruct((TBL,), jnp.int32),
        jax.ShapeDtypeStruct((E // 128, 128), jnp.int32),
        jax.ShapeDtypeStruct((E // 128, 128), jnp.int32),
    ],
    scratch_types=[
        pltpu.VMEM((CH,), jnp.int32),
        pltpu.VMEM((CH,), jnp.int32),
        pltpu.VMEM((C128, 128), jnp.int32),
        pltpu.VMEM((C128, 128), jnp.int32),
        pltpu.VMEM((C128, 128), jnp.int32),
        pltpu.SemaphoreType.DMA,
    ],
)
def _phase1(src_hbm, tgt_hbm, table_hbm, keys_hbm, rkeys_hbm,
            s_v, t_v, key_v, rkey_v, ids_v, sem):
    wid = lax.axis_index("s") * NC + lax.axis_index("c")
    base = wid * CH
    pltpu.sync_copy(src_hbm.at[pl.ds(base, CH)], s_v)
    pltpu.sync_copy(tgt_hbm.at[pl.ds(base, CH)], t_v)
    iot = lax.iota(jnp.int32, 16)

    def row(c, _):
        for l in range(8):
            off = c * 128 + l * 16
            s16 = s_v[pl.ds(off, 16)]
            t16 = t_v[pl.ds(off, 16)]
            key_v[c, pl.ds(l * 16, 16)] = (s16 << 12) | t16
            rkey_v[c, pl.ds(l * 16, 16)] = (t16 << 12) | s16
            ids_v[c, pl.ds(l * 16, 16)] = (
                jnp.full((16,), base + off, jnp.int32) + iot)
        return 0

    lax.fori_loop(0, C128, row, 0)

    pltpu.sync_copy(key_v, keys_hbm.at[pl.ds(wid * C128, C128)])
    pltpu.sync_copy(rkey_v, rkeys_hbm.at[pl.ds(wid * C128, C128)])

    for g in range(C128 // KG):
        hs = [pltpu.async_copy(ids_v.at[g * KG + i],
                               table_hbm.at[key_v.at[g * KG + i]], sem)
              for i in range(KG)]
        for h in hs:
            h.wait()


@functools.partial(
    pl.kernel,
    mesh=_mesh(),
    out_type=[
        jax.ShapeDtypeStruct((NC, N_NODES), jnp.float32),
        jax.ShapeDtypeStruct((NW, 48), jnp.float32),
    ],
    scratch_types=[
        pltpu.VMEM((CH,), jnp.int32),
        pltpu.VMEM((CH,), jnp.int32),
        pltpu.VMEM((CH,), jnp.int32),
        pltpu.VMEM((CH,), jnp.int32),
        pltpu.VMEM((CH,), jnp.int32),
        pltpu.VMEM((CH,), jnp.float32),
        pltpu.VMEM((CH,), jnp.float32),
        pltpu.VMEM((CH,), jnp.float32),
        pltpu.VMEM((N_NODES,), jnp.float32),
        pltpu.VMEM_SHARED((N_NODES,), jnp.float32),
        pltpu.VMEM((48,), jnp.float32),
        pltpu.SemaphoreType.DMA,
    ],
)
def _phase2(table_hbm, keys_hbm, rkeys_hbm, tgt_hbm,
            p0_hbm, p1_hbm, p2_hbm, p3_hbm, pseq_hbm, ppar_hbm,
            tree_hbm, scal_hbm,
            rk_v, t_v, j_v, js_v, kj_v, m_v, fw_v, rv_v, z_v,
            shared, s48_v, sem):
    cid = lax.axis_index("c")
    sid = lax.axis_index("s")
    wid = sid * NC + cid
    base = wid * CH

    pltpu.sync_copy(rkeys_hbm.at[pl.ds(base, CH)], rk_v)
    pltpu.sync_copy(tgt_hbm.at[pl.ds(base, CH)], t_v)

    def gather_chunks(src_hbm_full, idx_v, dst_v):
        for g in range(C128 // KG):
            hs = [pltpu.async_copy(
                src_hbm_full.at[idx_v.at[pl.ds((g * KG + i) * 128, 128)]],
                dst_v.at[pl.ds((g * KG + i) * 128, 128)], sem)
                for i in range(KG)]
            for h in hs:
                h.wait()

    gather_chunks(table_hbm, rk_v, j_v)

    def clamp(k, _):
        j16 = j_v[pl.ds(k * 16, 16)]
        js_v[pl.ds(k * 16, 16)] = jnp.minimum(jnp.maximum(j16, 0), E - 1)
        return 0

    lax.fori_loop(0, CH // 16, clamp, 0)

    gather_chunks(keys_hbm, js_v, kj_v)

    def vmask(k, cnt):
        j16 = j_v[pl.ds(k * 16, 16)]
        kj16 = kj_v[pl.ds(k * 16, 16)]
        rk16 = rk_v[pl.ds(k * 16, 16)]
        u = j16 >> 18
        d = (kj16 ^ rk16) | u
        nz = d | (0 - d)
        valid = (nz >> 31) + 1
        m_v[pl.ds(k * 16, 16)] = valid.astype(jnp.float32)
        return cnt + valid

    cnt16 = lax.fori_loop(0, CH // 16, vmask, jnp.zeros((16,), jnp.int32))

    def z(i, _):
        z_v[pl.ds(i * 16, 16)] = jnp.zeros((16,), jnp.float32)
        return 0

    lax.fori_loop(0, N_NODES // 16, z, 0)

    def col_pass(plane_hbm, acc):
        pltpu.sync_copy(plane_hbm.at[pl.ds(base, CH)], fw_v)
        gather_chunks(plane_hbm, js_v, rv_v)

        def prod(k, a):
            f16 = fw_v[pl.ds(k * 16, 16)]
            r16 = rv_v[pl.ds(k * 16, 16)]
            m16 = m_v[pl.ds(k * 16, 16)]
            return a + f16 * r16 * m16

        return lax.fori_loop(0, CH // 16, prod, acc)

    accA = jnp.zeros((16,), jnp.float32)
    for plane in (p0_hbm, p1_hbm, p2_hbm, p3_hbm):
        accA = col_pass(plane, accA)
    accD = col_pass(pseq_hbm, jnp.zeros((16,), jnp.float32))

    pltpu.sync_copy(ppar_hbm.at[pl.ds(base, CH)], fw_v)

    @pl.when(sid == 0)
    def _():
        pltpu.sync_copy(z_v, shared)

    plsc.subcore_barrier()

    for g in range(C128 // KG):
        hs = [pltpu.async_copy(
            fw_v.at[pl.ds((g * KG + i) * 128, 128)],
            shared.at[t_v.at[pl.ds((g * KG + i) * 128, 128)]],
            sem, add=True)
            for i in range(KG)]
        for h in hs:
            h.wait()

    plsc.subcore_barrier()

    @pl.when(sid == 0)
    def _():
        pltpu.sync_copy(shared, tree_hbm.at[cid])

    s48_v[pl.ds(0, 16)] = accA
    s48_v[pl.ds(16, 16)] = accD
    s48_v[pl.ds(32, 16)] = cnt16.astype(jnp.float32)
    pltpu.sync_copy(s48_v, scal_hbm.at[wid])


def _combine_body(tree_ref, scal_ref, o_total, o_anti, o_tree, o_dag):
    ps = jnp.sum(tree_ref[...], axis=0, keepdims=True)
    tree_loss = jnp.mean(jax.nn.softplus(ps - 1.0))
    A = jnp.sum(scal_ref[:, 0:16])
    D = jnp.sum(scal_ref[:, 16:32])
    cnt = jnp.sum(scal_ref[:, 32:48])
    anti = A / jnp.maximum(cnt * 4.0, 1.0)
    dag = D / jnp.maximum(cnt, 1.0)
    total = anti + tree_loss + 0.5 * dag
    o_total[0, 0] = total
    o_anti[0, 0] = anti
    o_tree[0, 0] = tree_loss
    o_dag[0, 0] = dag


_combine = pl.pallas_call(
    _combine_body,
    out_shape=[jax.ShapeDtypeStruct((1, 1), jnp.float32)] * 4,
    out_specs=[pl.BlockSpec(memory_space=pltpu.SMEM)] * 4,
)


def kernel(rel_probs, edge_index, num_nodes):
    del num_nodes
    src = edge_index[0]
    tgt = edge_index[1]
    p0, p1, p2, p3, pseq, ppar = (
        p.reshape(E) for p in _prep(rel_probs.reshape(E // 128, 128, NREL)))
    table, keys2, rkeys2 = _phase1(src, tgt)
    tree_part, scal_part = _phase2(
        table, keys2.reshape(E), rkeys2.reshape(E), tgt,
        p0, p1, p2, p3, pseq, ppar)
    total, anti, tree, dag = _combine(tree_part, scal_part)
    return (total.reshape(()), anti.reshape(()), tree.reshape(()),
            dag.reshape(()))


# speedup vs baseline: 3.2555x; 1.0056x over previous
import functools

import jax
import jax.numpy as jnp
from jax import lax
from jax.experimental import pallas as pl
from jax.experimental.pallas import tpu as pltpu
from jax.experimental.pallas import tpu_sc as plsc

N_NODES = 4096
E = 262144
NREL = 8
PARENT = 4
SEQUENCE = 6

NC = 2
NS = 16
NW = NC * NS
CH = E // NW
C128 = CH // 128
KG = 16
TBL = N_NODES * N_NODES


def _mesh():
    return plsc.VectorSubcoreMesh(core_axis_name="c", subcore_axis_name="s")


_PREP_COLS = (0, 1, 2, 3, SEQUENCE, PARENT)


def _prep_body(rel_ref, *out_refs):
    x = rel_ref[...]
    for c, o in zip(_PREP_COLS, out_refs):
        o[...] = x[:, :, c]


_prep = pl.pallas_call(
    _prep_body,
    grid=(E // (8 * 128),),
    in_specs=[pl.BlockSpec((8, 128, NREL), lambda i: (i, 0, 0))],
    out_specs=[pl.BlockSpec((8, 128), lambda i: (i, 0))] * len(_PREP_COLS),
    out_shape=[jax.ShapeDtypeStruct((E // 128, 128), jnp.float32)]
    * len(_PREP_COLS),
)


@functools.partial(
    pl.kernel,
    mesh=_mesh(),
    out_type=[
        jax.ShapeDtypeStruct((TBL,), jnp.int32),
        jax.ShapeDtypeStruct((E,), jnp.int32),
        jax.ShapeDtypeStruct((E,), jnp.int32),
    ],
    scratch_types=[
        pltpu.VMEM((CH,), jnp.int32),
        pltpu.VMEM((CH,), jnp.int32),
        pltpu.VMEM((CH,), jnp.int32),
        pltpu.VMEM((CH,), jnp.int32),
        pltpu.VMEM((CH,), jnp.int32),
        pltpu.SemaphoreType.DMA,
    ],
)
def _phase1(src_hbm, tgt_hbm, table_hbm, keys_hbm, rkeys_hbm,
            s_v, t_v, key_v, rkey_v, ids_v, sem):
    wid = lax.axis_index("s") * NC + lax.axis_index("c")
    base = wid * CH
    pltpu.sync_copy(src_hbm.at[pl.ds(base, CH)], s_v)
    pltpu.sync_copy(tgt_hbm.at[pl.ds(base, CH)], t_v)
    iot = lax.iota(jnp.int32, 16)

    def row(k, _):
        off = k * 16
        s16 = s_v[pl.ds(off, 16)]
        t16 = t_v[pl.ds(off, 16)]
        key_v[pl.ds(off, 16)] = (s16 << 12) | t16
        rkey_v[pl.ds(off, 16)] = (t16 << 12) | s16
        ids_v[pl.ds(off, 16)] = jnp.full((16,), base + off, jnp.int32) + iot
        return 0

    lax.fori_loop(0, CH // 16, row, 0)

    pltpu.sync_copy(key_v, keys_hbm.at[pl.ds(base, CH)])
    pltpu.sync_copy(rkey_v, rkeys_hbm.at[pl.ds(base, CH)])

    pltpu.async_copy(ids_v, table_hbm.at[key_v], sem).wait()


@functools.partial(
    pl.kernel,
    mesh=_mesh(),
    out_type=[
        jax.ShapeDtypeStruct((NC, N_NODES), jnp.float32),
        jax.ShapeDtypeStruct((NW, 48), jnp.float32),
    ],
    scratch_types=[
        pltpu.VMEM((CH,), jnp.int32),
        pltpu.VMEM((CH,), jnp.int32),
        pltpu.VMEM((CH,), jnp.int32),
        pltpu.VMEM((CH,), jnp.int32),
        pltpu.VMEM((CH,), jnp.float32),
        pltpu.VMEM((CH,), jnp.float32),
        pltpu.VMEM((CH,), jnp.float32),
        pltpu.VMEM((CH,), jnp.float32),
        pltpu.VMEM((CH,), jnp.float32),
        pltpu.VMEM((CH,), jnp.float32),
        pltpu.VMEM((CH,), jnp.float32),
        pltpu.VMEM((CH,), jnp.float32),
        pltpu.VMEM((CH,), jnp.float32),
        pltpu.VMEM((CH,), jnp.float32),
        pltpu.VMEM((CH,), jnp.float32),
        pltpu.VMEM((N_NODES,), jnp.float32),
        pltpu.VMEM_SHARED((N_NODES,), jnp.float32),
        pltpu.VMEM((48,), jnp.float32),
        pltpu.SemaphoreType.DMA,
    ],
)
def _phase2(table_hbm, keys_hbm, rkeys_hbm, tgt_hbm,
            p0_hbm, p1_hbm, p2_hbm, p3_hbm, pseq_hbm, ppar_hbm,
            tree_hbm, scal_hbm,
            rk_v, js_v, kj_v, t_v,
            f0_v, f1_v, f2_v, f3_v, fs_v, fp_v,
            r0_v, r1_v, r2_v, r3_v, rs_v,
            z_v, shared, s48_v, sem):
    cid = lax.axis_index("c")
    sid = lax.axis_index("s")
    wid = sid * NC + cid
    base = wid * CH
    lin = pl.ds(base, CH)

    def z(i, _):
        z_v[pl.ds(i * 16, 16)] = jnp.zeros((16,), jnp.float32)
        return 0

    lax.fori_loop(0, N_NODES // 16, z, 0)

    @pl.when(sid == 0)
    def _():
        pltpu.sync_copy(z_v, shared)

    pltpu.sync_copy(rkeys_hbm.at[lin], rk_v)
    pltpu.sync_copy(tgt_hbm.at[lin], t_v)
    for plane, dst in ((p0_hbm, f0_v), (p1_hbm, f1_v), (p2_hbm, f2_v),
                       (p3_hbm, f3_v), (pseq_hbm, fs_v), (ppar_hbm, fp_v)):
        pltpu.sync_copy(plane.at[lin], dst)

    pltpu.async_copy(table_hbm.at[rk_v], js_v, sem).wait()

    def clamp(k, _):
        j16 = js_v[pl.ds(k * 16, 16)]
        js_v[pl.ds(k * 16, 16)] = jnp.minimum(jnp.maximum(j16, 0), E - 1)
        return 0

    lax.fori_loop(0, CH // 16, clamp, 0)

    hs = [pltpu.async_copy(keys_hbm.at[js_v], kj_v, sem)]
    hs += [pltpu.async_copy(p.at[js_v], d, sem)
           for p, d in ((p0_hbm, r0_v), (p1_hbm, r1_v), (p2_hbm, r2_v),
                        (p3_hbm, r3_v), (pseq_hbm, rs_v))]
    for h in hs:
        h.wait()

    def prod(k, acc):
        aA, aD, cnt = acc
        s = pl.ds(k * 16, 16)
        d = kj_v[s] ^ rk_v[s]
        nz = d | (0 - d)
        valid = (nz >> 31) + 1
        m16 = valid.astype(jnp.float32)
        aA = aA + (f0_v[s] * r0_v[s] + f1_v[s] * r1_v[s]
                   + f2_v[s] * r2_v[s] + f3_v[s] * r3_v[s]) * m16
        aD = aD + fs_v[s] * rs_v[s] * m16
        return (aA, aD, cnt + valid)

    zf = jnp.zeros((16,), jnp.float32)
    accA, accD, cnt16 = lax.fori_loop(
        0, CH // 16, prod, (zf, zf, jnp.zeros((16,), jnp.int32)))

    plsc.subcore_barrier()
    pltpu.async_copy(fp_v, shared.at[t_v], sem, add=True).wait()
    plsc.subcore_barrier()

    @pl.when(sid == 0)
    def _():
        pltpu.sync_copy(shared, tree_hbm.at[cid])

    s48_v[pl.ds(0, 16)] = accA
    s48_v[pl.ds(16, 16)] = accD
    s48_v[pl.ds(32, 16)] = cnt16.astype(jnp.float32)
    pltpu.sync_copy(s48_v, scal_hbm.at[wid])


def _combine_body(tree_ref, scal_ref, o_total, o_anti, o_tree, o_dag):
    ps = jnp.sum(tree_ref[...], axis=0, keepdims=True)
    tree_loss = jnp.mean(jax.nn.softplus(ps - 1.0))
    A = jnp.sum(scal_ref[:, 0:16])
    D = jnp.sum(scal_ref[:, 16:32])
    cnt = jnp.sum(scal_ref[:, 32:48])
    anti = A / jnp.maximum(cnt * 4.0, 1.0)
    dag = D / jnp.maximum(cnt, 1.0)
    total = anti + tree_loss + 0.5 * dag
    o_total[0, 0] = total
    o_anti[0, 0] = anti
    o_tree[0, 0] = tree_loss
    o_dag[0, 0] = dag


_combine = pl.pallas_call(
    _combine_body,
    out_shape=[jax.ShapeDtypeStruct((1, 1), jnp.float32)] * 4,
    out_specs=[pl.BlockSpec(memory_space=pltpu.SMEM)] * 4,
)


def kernel(rel_probs, edge_index, num_nodes):
    del num_nodes
    src = edge_index[0]
    tgt = edge_index[1]
    planes = _prep(rel_probs.reshape(E // 128, 128, NREL))
    p0, p1, p2, p3, pseq, ppar = (p.reshape(E) for p in planes)
    table, keys, rkeys = _phase1(src, tgt)
    tree_part, scal_part = _phase2(
        table, keys, rkeys, tgt, p0, p1, p2, p3, pseq, ppar)
    total, anti, tree, dag = _combine(tree_part, scal_part)
    return (total.reshape(()), anti.reshape(()), tree.reshape(()),
            dag.reshape(()))
